# baseline (speedup 1.0000x reference)
import jax
import jax.numpy as jnp
from jax import lax
from jax.experimental import pallas as pl
from jax.experimental.pallas import tpu as pltpu
from jax.experimental.pallas import tpu_sc as plsc

N, E, M, C, D = 8192, 50000, 65536, 1024, 1024
NC, NS, L = 2, 16, 16
NW = NC * NS
RPW = N // NW
NP = 4
RPP = RPW // NP
ACC_R = RPP + 1
CHK = 2048
NCHK = M // CHK
CAP = 7680
SEL = CHK + 4 * L
BE = 16


def _sc_body(ent_hbm, ee_hbm, es_hbm, ctx_hbm, cnt_hbm,
             acc_v, rows_a, rows_b, ee_c, es_c, own_ee, own_es,
             sel_ee, sel_loc, hist_v, sem_a, sem_b):
    c0 = lax.axis_index("c")
    s = lax.axis_index("s")
    w = c0 * NS + s
    wrow0 = w * RPW

    zv = jnp.zeros((L,), jnp.float32)

    def flat_zero(i):
        acc_v[i // D, pl.ds(i % D, L)] = zv
    ones_f = jnp.ones((L,), jnp.float32)
    neg1 = jnp.full((L,), -1, jnp.int32)
    garb_loc = jnp.full((L,), RPP, jnp.int32)
    zero_i = jnp.zeros((L,), jnp.int32)

    @plsc.parallel_loop(0, CAP + L, step=L, unroll=4)
    def _(i):
        own_es[pl.ds(i, L)] = neg1

    def build_chunk(q, cnt):
        pltpu.sync_copy(ee_hbm.at[pl.ds(q * CHK, CHK)], ee_c)
        pltpu.sync_copy(es_hbm.at[pl.ds(q * CHK, CHK)], es_c)

        def gb(i, cnt2):
            esv = es_c[pl.ds(i * L, L)]
            eev = ee_c[pl.ds(i * L, L)]
            rel = esv - wrow0
            msk = (rel >= 0) & (rel < RPW)
            plsc.store_compressed(own_es.at[pl.ds(cnt2, L)], esv, mask=msk)
            plsc.store_compressed(own_ee.at[pl.ds(cnt2, L)], eev, mask=msk)
            return cnt2 + plsc.all_reduce_population_count(msk)[0]

        return lax.cond(cnt <= CAP - CHK,
                        lambda c: lax.fori_loop(0, CHK // L, gb, c),
                        lambda c: jnp.int32(CAP + 1), cnt)

    cnt_own = lax.fori_loop(0, NCHK, build_chunk, jnp.int32(0))
    fast = cnt_own <= CAP

    def start_gather(j, buf, sem):
        pltpu.async_copy(ent_hbm.at[sel_ee.at[pl.ds(j * BE, BE)]], buf, sem)

    def wait_gather(buf, sem):
        pltpu.make_async_copy(ent_hbm.at[pl.ds(0, BE)], buf, sem).wait()

    def do_adds(j, buf):
        for g in range(BE // L):
            locv = sel_loc[pl.ds(j * BE + g * L, L)]
            plsc.addupdate_scatter(hist_v, [locv], ones_f)
            for k in range(L):
                loc = locv[k]

                @plsc.parallel_loop(0, D, step=L, unroll=8)
                def _(cc):
                    plsc.addupdate(acc_v.at[loc, pl.ds(cc, L)],
                                   buf[g * L + k, pl.ds(cc, L)])

    def compact_and_add(src_es, src_ee, base, ngroups, row0):
        def cb(i, scnt):
            esv = src_es[pl.ds(base + i * L, L)]
            eev = src_ee[pl.ds(base + i * L, L)]
            rel = esv - row0
            msk = (rel >= 0) & (rel < RPP)
            plsc.store_compressed(sel_loc.at[pl.ds(scnt, L)], rel, mask=msk)
            plsc.store_compressed(sel_ee.at[pl.ds(scnt, L)], eev, mask=msk)
            return scnt + plsc.all_reduce_population_count(msk)[0]

        scnt = lax.fori_loop(0, ngroups, cb, jnp.int32(0))
        for t in range(2 * BE // L):
            sel_loc[pl.ds(scnt + t * L, L)] = garb_loc
            sel_ee[pl.ds(scnt + t * L, L)] = zero_i
        npair = (scnt + (2 * BE - 1)) // (2 * BE)

        @pl.when(npair > 0)
        def _():
            start_gather(0, rows_a, sem_a)

            def pair(p, _):
                j0 = 2 * p
                start_gather(j0 + 1, rows_b, sem_b)
                wait_gather(rows_a, sem_a)
                do_adds(j0, rows_a)

                @pl.when(p + 1 < npair)
                def _():
                    start_gather(j0 + 2, rows_a, sem_a)

                wait_gather(rows_b, sem_b)
                do_adds(j0 + 1, rows_b)
                return 0

            lax.fori_loop(0, npair, pair, 0)

    def start_pass():
        @plsc.parallel_loop(0, ACC_R * D, step=L, unroll=8)
        def _(i):
            flat_zero(i)

        @pl.loop(0, ACC_R, step=L)
        def _(i):
            hist_v[pl.ds(i, L)] = zv

    def end_pass(row0):
        pltpu.sync_copy(acc_v.at[pl.ds(0, RPP)], ctx_hbm.at[pl.ds(row0, RPP)])
        pltpu.sync_copy(hist_v.at[pl.ds(0, RPP)], cnt_hbm.at[pl.ds(row0, RPP)])

    @pl.when(fast)
    def _():
        nq = (cnt_own + (CHK - 1)) // CHK

        @pl.loop(0, NP)
        def _(p):
            row0 = wrow0 + p * RPP
            start_pass()

            def fq(qi, _):
                compact_and_add(own_es, own_ee, qi * CHK, CHK // L, row0)
                return 0

            lax.fori_loop(0, nq, fq, 0)
            end_pass(row0)

    @pl.when(jnp.logical_not(fast))
    def _():
        @pl.loop(0, NP)
        def _(p):
            row0 = wrow0 + p * RPP
            start_pass()

            def sq(q, _):
                pltpu.sync_copy(ee_hbm.at[pl.ds(q * CHK, CHK)], ee_c)
                pltpu.sync_copy(es_hbm.at[pl.ds(q * CHK, CHK)], es_c)
                compact_and_add(es_c, ee_c, 0, CHK // L, row0)
                return 0

            lax.fori_loop(0, NCHK, sq, 0)
            end_pass(row0)


def _sc_segment_sum(entity_memory, edge_entity, edge_sent):
    mesh = plsc.VectorSubcoreMesh(core_axis_name="c", subcore_axis_name="s",
                                  num_cores=NC, num_subcores=NS)
    fn = pl.kernel(
        _sc_body,
        out_type=(jax.ShapeDtypeStruct((N, D), jnp.float32),
                  jax.ShapeDtypeStruct((N,), jnp.float32)),
        mesh=mesh,
        scratch_types=[
            pltpu.VMEM((ACC_R, D), jnp.float32),
            pltpu.VMEM((BE, D), jnp.float32),
            pltpu.VMEM((BE, D), jnp.float32),
            pltpu.VMEM((CHK,), jnp.int32),
            pltpu.VMEM((CHK,), jnp.int32),
            pltpu.VMEM((CAP + L,), jnp.int32),
            pltpu.VMEM((CAP + L,), jnp.int32),
            pltpu.VMEM((SEL,), jnp.int32),
            pltpu.VMEM((SEL,), jnp.int32),
            pltpu.VMEM((ACC_R,), jnp.float32),
            pltpu.SemaphoreType.DMA,
            pltpu.SemaphoreType.DMA,
        ],
        compiler_params=pltpu.CompilerParams(needs_layout_passes=False),
    )
    return fn(entity_memory, edge_entity, edge_sent)


BLK = 256


def _tc_pre_body(sv_ref, wt_ref, cidx_ref, l1_ref, m_ref):
    i = pl.program_id(0)
    l1_ref[...] = jnp.dot(sv_ref[...].astype(jnp.bfloat16), wt_ref[...],
                          preferred_element_type=jnp.float32)
    rows = i * BLK + lax.broadcasted_iota(jnp.int32, (BLK, C), 0)
    mask = jnp.any(rows == cidx_ref[...], axis=1, keepdims=True)
    m_ref[...] = mask.astype(jnp.float32)


def _tc_pre(sv, wt1, conflict_idx):
    cidx2 = conflict_idx.reshape(1, C)
    return pl.pallas_call(
        _tc_pre_body,
        grid=(N // BLK,),
        in_specs=[
            pl.BlockSpec((BLK, D), lambda i: (i, 0)),
            pl.BlockSpec((D, D), lambda i: (0, 0)),
            pl.BlockSpec((1, C), lambda i: (0, 0)),
        ],
        out_specs=(pl.BlockSpec((BLK, D), lambda i: (i, 0)),
                   pl.BlockSpec((BLK, 1), lambda i: (i, 0))),
        out_shape=(jax.ShapeDtypeStruct((N, D), jnp.float32),
                   jax.ShapeDtypeStruct((N, 1), jnp.float32)),
    )(sv, wt1, cidx2)


def _tc_post_body(cnt_ref, sv_ref, ctx_ref, l1_ref, m_ref, wt_ref, b_ref,
                  cb_ref, out_ref):
    cnt = cnt_ref[...]
    ctx = ctx_ref[...] / jnp.maximum(cnt, 1.0)
    sv = sv_ref[...]
    logits = l1_ref[...] + jnp.dot(ctx.astype(jnp.bfloat16), wt_ref[...],
                                   preferred_element_type=jnp.float32)
    gate = jax.nn.sigmoid(logits + b_ref[...])
    refined = sv + jnp.where(cnt > 0.0, gate * ctx, 0.0)
    out_ref[...] = refined + jnp.where(m_ref[...] > 0.0, cb_ref[...], 0.0)


def _tc_fuse(counts, sv, ctx_sum, l1, mask, wt2, b, conflict_bias):
    cnt2 = counts.reshape(N, 1)
    b2 = b.reshape(1, D)
    cb2 = conflict_bias.reshape(1, D)
    return pl.pallas_call(
        _tc_post_body,
        grid=(N // BLK,),
        in_specs=[
            pl.BlockSpec((BLK, 1), lambda i: (i, 0)),
            pl.BlockSpec((BLK, D), lambda i: (i, 0)),
            pl.BlockSpec((BLK, D), lambda i: (i, 0)),
            pl.BlockSpec((BLK, D), lambda i: (i, 0)),
            pl.BlockSpec((BLK, 1), lambda i: (i, 0)),
            pl.BlockSpec((D, D), lambda i: (0, 0)),
            pl.BlockSpec((1, D), lambda i: (0, 0)),
            pl.BlockSpec((1, D), lambda i: (0, 0)),
        ],
        out_specs=pl.BlockSpec((BLK, D), lambda i: (i, 0)),
        out_shape=jax.ShapeDtypeStruct((N, D), jnp.float32),
    )(cnt2, sv, ctx_sum, l1, mask, wt2, b2, cb2)


def kernel(sentence_vectors, entity_memory, W, b, conflict_bias,
           edge_entity, edge_sent, conflict_idx):
    wt = W.T.astype(jnp.bfloat16)
    ctx_sum, counts = _sc_segment_sum(entity_memory, edge_entity, edge_sent)
    l1, mask = _tc_pre(sentence_vectors, wt[:D], conflict_idx)
    return _tc_fuse(counts, sentence_vectors, ctx_sum, l1, mask, wt[D:], b,
                    conflict_bias)


# speedup vs baseline: 1.0737x; 1.0737x over previous
import jax
import jax.numpy as jnp
from jax import lax
from jax.experimental import pallas as pl
from jax.experimental.pallas import tpu as pltpu
from jax.experimental.pallas import tpu_sc as plsc

N, E, M, C, D = 8192, 50000, 65536, 1024, 1024
NC, NS, L = 2, 16, 16
NW = NC * NS
RPW = N // NW
NP = 4
RPP = RPW // NP
ACC_R = RPP + 1
CHK = 2048
NCHK = M // CHK
CAP = 7680
SEL = CHK + 4 * L
BE = 16


def _sc_body(ent_hbm, ee_hbm, es_hbm, ctx_hbm, cnt_hbm,
             acc_v, rows_a, rows_b, ee_c, es_c, own_ee, own_es,
             sel_ee, sel_loc, hist_v, sem_a, sem_b):
    c0 = lax.axis_index("c")
    s = lax.axis_index("s")
    w = c0 * NS + s
    wrow0 = w * RPW

    zv = jnp.zeros((L,), jnp.float32)

    def flat_zero(i):
        acc_v[i // D, pl.ds(i % D, L)] = zv
    ones_f = jnp.ones((L,), jnp.float32)
    neg1 = jnp.full((L,), -1, jnp.int32)
    garb_loc = jnp.full((L,), RPP, jnp.int32)
    zero_i = jnp.zeros((L,), jnp.int32)

    @plsc.parallel_loop(0, CAP + L, step=L, unroll=4)
    def _(i):
        own_es[pl.ds(i, L)] = neg1

    def build_chunk(q, cnt):
        pltpu.sync_copy(ee_hbm.at[pl.ds(q * CHK, CHK)], ee_c)
        pltpu.sync_copy(es_hbm.at[pl.ds(q * CHK, CHK)], es_c)

        def scan_chunk(c):
            @plsc.parallel_loop(0, CHK // L, unroll=4, carry=c)
            def cnt2(i, cnt2):
                esv = es_c[pl.ds(i * L, L)]
                eev = ee_c[pl.ds(i * L, L)]
                rel = esv - wrow0
                msk = (rel >= 0) & (rel < RPW)
                plsc.store_compressed(own_es.at[pl.ds(cnt2, L)], esv,
                                      mask=msk)
                plsc.store_compressed(own_ee.at[pl.ds(cnt2, L)], eev,
                                      mask=msk)
                return cnt2 + plsc.all_reduce_population_count(msk)[0]

            return cnt2

        return lax.cond(cnt <= CAP - CHK, scan_chunk,
                        lambda c: jnp.int32(CAP + 1), cnt)

    cnt_own = lax.fori_loop(0, NCHK, build_chunk, jnp.int32(0))
    fast = cnt_own <= CAP

    def start_gather(j, buf, sem):
        pltpu.async_copy(ent_hbm.at[sel_ee.at[pl.ds(j * BE, BE)]], buf, sem)

    def wait_gather(buf, sem):
        pltpu.make_async_copy(ent_hbm.at[pl.ds(0, BE)], buf, sem).wait()

    def do_adds(j, buf):
        for g in range(BE // L):
            locv = sel_loc[pl.ds(j * BE + g * L, L)]
            plsc.addupdate_scatter(hist_v, [locv], ones_f)
            for k in range(L):
                loc = locv[k]

                @plsc.parallel_loop(0, D, step=L, unroll=8)
                def _(cc):
                    plsc.addupdate(acc_v.at[loc, pl.ds(cc, L)],
                                   buf[g * L + k, pl.ds(cc, L)])

    def compact_and_add(src_es, src_ee, base, ngroups, row0):
        @plsc.parallel_loop(0, ngroups, unroll=4, carry=jnp.int32(0))
        def scnt(i, scnt):
            esv = src_es[pl.ds(base + i * L, L)]
            eev = src_ee[pl.ds(base + i * L, L)]
            rel = esv - row0
            msk = (rel >= 0) & (rel < RPP)
            plsc.store_compressed(sel_loc.at[pl.ds(scnt, L)], rel, mask=msk)
            plsc.store_compressed(sel_ee.at[pl.ds(scnt, L)], eev, mask=msk)
            return scnt + plsc.all_reduce_population_count(msk)[0]
        for t in range(2 * BE // L):
            sel_loc[pl.ds(scnt + t * L, L)] = garb_loc
            sel_ee[pl.ds(scnt + t * L, L)] = zero_i
        npair = (scnt + (2 * BE - 1)) // (2 * BE)

        @pl.when(npair > 0)
        def _():
            start_gather(0, rows_a, sem_a)

            def pair(p, _):
                j0 = 2 * p
                start_gather(j0 + 1, rows_b, sem_b)
                wait_gather(rows_a, sem_a)
                do_adds(j0, rows_a)

                @pl.when(p + 1 < npair)
                def _():
                    start_gather(j0 + 2, rows_a, sem_a)

                wait_gather(rows_b, sem_b)
                do_adds(j0 + 1, rows_b)
                return 0

            lax.fori_loop(0, npair, pair, 0)

    def start_pass():
        @plsc.parallel_loop(0, ACC_R * D, step=L, unroll=8)
        def _(i):
            flat_zero(i)

        @pl.loop(0, ACC_R, step=L)
        def _(i):
            hist_v[pl.ds(i, L)] = zv

    def end_pass(row0):
        pltpu.sync_copy(acc_v.at[pl.ds(0, RPP)], ctx_hbm.at[pl.ds(row0, RPP)])
        pltpu.sync_copy(hist_v.at[pl.ds(0, RPP)], cnt_hbm.at[pl.ds(row0, RPP)])

    @pl.when(fast)
    def _():
        nq = (cnt_own + (CHK - 1)) // CHK

        @pl.loop(0, NP)
        def _(p):
            row0 = wrow0 + p * RPP
            start_pass()

            def fq(qi, _):
                compact_and_add(own_es, own_ee, qi * CHK, CHK // L, row0)
                return 0

            lax.fori_loop(0, nq, fq, 0)
            end_pass(row0)

    @pl.when(jnp.logical_not(fast))
    def _():
        @pl.loop(0, NP)
        def _(p):
            row0 = wrow0 + p * RPP
            start_pass()

            def sq(q, _):
                pltpu.sync_copy(ee_hbm.at[pl.ds(q * CHK, CHK)], ee_c)
                pltpu.sync_copy(es_hbm.at[pl.ds(q * CHK, CHK)], es_c)
                compact_and_add(es_c, ee_c, 0, CHK // L, row0)
                return 0

            lax.fori_loop(0, NCHK, sq, 0)
            end_pass(row0)


def _sc_segment_sum(entity_memory, edge_entity, edge_sent):
    mesh = plsc.VectorSubcoreMesh(core_axis_name="c", subcore_axis_name="s",
                                  num_cores=NC, num_subcores=NS)
    fn = pl.kernel(
        _sc_body,
        out_type=(jax.ShapeDtypeStruct((N, D), jnp.float32),
                  jax.ShapeDtypeStruct((N,), jnp.float32)),
        mesh=mesh,
        scratch_types=[
            pltpu.VMEM((ACC_R, D), jnp.float32),
            pltpu.VMEM((BE, D), jnp.float32),
            pltpu.VMEM((BE, D), jnp.float32),
            pltpu.VMEM((CHK,), jnp.int32),
            pltpu.VMEM((CHK,), jnp.int32),
            pltpu.VMEM((CAP + L,), jnp.int32),
            pltpu.VMEM((CAP + L,), jnp.int32),
            pltpu.VMEM((SEL,), jnp.int32),
            pltpu.VMEM((SEL,), jnp.int32),
            pltpu.VMEM((ACC_R,), jnp.float32),
            pltpu.SemaphoreType.DMA,
            pltpu.SemaphoreType.DMA,
        ],
        compiler_params=pltpu.CompilerParams(needs_layout_passes=False),
    )
    return fn(entity_memory, edge_entity, edge_sent)


BLK = 256


def _tc_pre_body(sv_ref, wt_ref, cidx_ref, l1_ref, m_ref):
    i = pl.program_id(0)
    l1_ref[...] = jnp.dot(sv_ref[...].astype(jnp.bfloat16), wt_ref[...],
                          preferred_element_type=jnp.float32)
    rows = i * BLK + lax.broadcasted_iota(jnp.int32, (BLK, C), 0)
    mask = jnp.any(rows == cidx_ref[...], axis=1, keepdims=True)
    m_ref[...] = mask.astype(jnp.float32)


def _tc_pre(sv, wt1, conflict_idx):
    cidx2 = conflict_idx.reshape(1, C)
    return pl.pallas_call(
        _tc_pre_body,
        grid=(N // BLK,),
        in_specs=[
            pl.BlockSpec((BLK, D), lambda i: (i, 0)),
            pl.BlockSpec((D, D), lambda i: (0, 0)),
            pl.BlockSpec((1, C), lambda i: (0, 0)),
        ],
        out_specs=(pl.BlockSpec((BLK, D), lambda i: (i, 0)),
                   pl.BlockSpec((BLK, 1), lambda i: (i, 0))),
        out_shape=(jax.ShapeDtypeStruct((N, D), jnp.float32),
                   jax.ShapeDtypeStruct((N, 1), jnp.float32)),
    )(sv, wt1, cidx2)


def _tc_post_body(cnt_ref, sv_ref, ctx_ref, l1_ref, m_ref, wt_ref, b_ref,
                  cb_ref, out_ref):
    cnt = cnt_ref[...]
    ctx = ctx_ref[...] / jnp.maximum(cnt, 1.0)
    sv = sv_ref[...]
    logits = l1_ref[...] + jnp.dot(ctx.astype(jnp.bfloat16), wt_ref[...],
                                   preferred_element_type=jnp.float32)
    gate = jax.nn.sigmoid(logits + b_ref[...])
    refined = sv + jnp.where(cnt > 0.0, gate * ctx, 0.0)
    out_ref[...] = refined + jnp.where(m_ref[...] > 0.0, cb_ref[...], 0.0)


def _tc_fuse(counts, sv, ctx_sum, l1, mask, wt2, b, conflict_bias):
    cnt2 = counts.reshape(N, 1)
    b2 = b.reshape(1, D)
    cb2 = conflict_bias.reshape(1, D)
    return pl.pallas_call(
        _tc_post_body,
        grid=(N // BLK,),
        in_specs=[
            pl.BlockSpec((BLK, 1), lambda i: (i, 0)),
            pl.BlockSpec((BLK, D), lambda i: (i, 0)),
            pl.BlockSpec((BLK, D), lambda i: (i, 0)),
            pl.BlockSpec((BLK, D), lambda i: (i, 0)),
            pl.BlockSpec((BLK, 1), lambda i: (i, 0)),
            pl.BlockSpec((D, D), lambda i: (0, 0)),
            pl.BlockSpec((1, D), lambda i: (0, 0)),
            pl.BlockSpec((1, D), lambda i: (0, 0)),
        ],
        out_specs=pl.BlockSpec((BLK, D), lambda i: (i, 0)),
        out_shape=jax.ShapeDtypeStruct((N, D), jnp.float32),
    )(cnt2, sv, ctx_sum, l1, mask, wt2, b2, cb2)


def kernel(sentence_vectors, entity_memory, W, b, conflict_bias,
           edge_entity, edge_sent, conflict_idx):
    wt = W.T.astype(jnp.bfloat16)
    ctx_sum, counts = _sc_segment_sum(entity_memory, edge_entity, edge_sent)
    l1, mask = _tc_pre(sentence_vectors, wt[:D], conflict_idx)
    return _tc_fuse(counts, sentence_vectors, ctx_sum, l1, mask, wt[D:], b,
                    conflict_bias)
